# baseline (speedup 1.0000x reference)
import functools

import jax
import jax.numpy as jnp
from jax import lax
from jax.experimental import pallas as pl
from jax.experimental.pallas import tpu as pltpu

NGEOM = 200
BSIZE = 64
NVALS = 2000000
BGEOM = 20
NSWEEP = 4


def _dgT00(a, b):
    return lax.dot_general(a, b, (((0,), (0,)), ((), ())),
                           preferred_element_type=jnp.float32)


def _dg11(a, b):
    return lax.dot_general(a, b, (((1,), (1,)), ((), ())),
                           preferred_element_type=jnp.float32)


def _coeffs(app, aqq, apq):
    zero = apq == 0.0
    apq_s = jnp.where(zero, 1.0, apq)
    tau = (aqq - app) / (2.0 * apq_s)
    t = jnp.sign(tau) / (jnp.abs(tau) + jnp.sqrt(tau * tau + 1.0))
    t = jnp.where(tau == 0.0, 1.0, t)
    t = jnp.where(zero, 0.0, t)
    c = lax.rsqrt(t * t + 1.0)
    s = t * c
    c = jnp.where(zero, 1.0, c)
    s = jnp.where(zero, 0.0, s)
    return c, s, t


def _perm_rows(A):
    return jnp.concatenate([A[:, 0:1], A[:, 32:33], A[:, 1:31],
                            A[:, 33:64], A[:, 31:32]], axis=1)


def _perm_cols(A):
    return jnp.concatenate([A[:, :, 0:1], A[:, :, 32:33], A[:, :, 1:31],
                            A[:, :, 33:64], A[:, :, 31:32]], axis=2)


def _masks():
    ks = lax.broadcasted_iota(jnp.int32, (1, BSIZE, BSIZE), 1)
    js = lax.broadcasted_iota(jnp.int32, (1, BSIZE, BSIZE), 2)
    eye = ks == js
    off = js == ks + 32
    return eye, off


def _rot_perm_rows(A, c_s, s_s):
    T, Bt = A[:, 0:32, :], A[:, 32:64, :]
    up = c_s * T - s_s * Bt
    dn = s_s * T + c_s * Bt
    return jnp.concatenate([up[:, 0:1], dn[:, 0:1], up[:, 1:31],
                            dn[:, 1:32], up[:, 31:32]], axis=1)


def _rot_perm_cols(A, c_l, s_l):
    L, R = A[:, :, 0:32], A[:, :, 32:64]
    lf = c_l * L - s_l * R
    rt = s_l * L + c_l * R
    return jnp.concatenate([lf[:, :, 0:1], rt[:, :, 0:1], lf[:, :, 1:31],
                            rt[:, :, 1:32], lf[:, :, 31:32]], axis=2)


def _perm_lanes(d):
    return jnp.concatenate([d[:, :, 0:1], d[:, :, 32:33], d[:, :, 1:31],
                            d[:, :, 33:64], d[:, :, 31:32]], axis=2)


def _jacobi_step(_, carry):
    A, V, d = carry
    ks = lax.broadcasted_iota(jnp.int32, A.shape, 1)
    js = lax.broadcasted_iota(jnp.int32, A.shape, 2)
    off = js == ks + 32
    ov_l = jnp.sum(jnp.where(off, A, jnp.zeros_like(A)), axis=1,
                   keepdims=True)
    apq = ov_l[:, :, 32:64]
    app, aqq = d[:, :, 0:32], d[:, :, 32:64]
    c_l, s_l, t_l = _coeffs(app, aqq, apq)
    d_new = jnp.concatenate([app - t_l * apq, aqq + t_l * apq], axis=2)
    d = _perm_lanes(d_new)
    eye32 = (lax.broadcasted_iota(jnp.int32, (32, 32), 0) ==
             lax.broadcasted_iota(jnp.int32, (32, 32), 1)
             ).astype(jnp.float32)
    cs_list = []
    for b in range(BGEOM):
        CS = jnp.concatenate([c_l[b], s_l[b]], axis=0)
        CST = lax.dot_general(eye32, CS, (((1,), (1,)), ((), ())),
                              preferred_element_type=jnp.float32)
        cs_list.append(CST[None])
    CSs = jnp.concatenate(cs_list, axis=0)
    c_s, s_s = CSs[:, :, 0:1], CSs[:, :, 1:2]
    A = _rot_perm_rows(A, c_s, s_s)
    A = _rot_perm_cols(A, c_l, s_l)
    V = _rot_perm_cols(V, c_l, s_l)
    return A, V, d


def _dense_body(S_ref, G_ref, rho_ref, qn_ref, phiS_ref, occ_ref, H_ref,
                out_ref):
    S = S_ref[...]
    G = G_ref[...]
    rho_in = rho_ref[...]
    qn = qn_ref[...]
    phiS = phiS_ref[...]
    occ = occ_ref[...]
    H = H_ref[...]

    qbasis = rho_in * S
    GOP = jnp.sum(qbasis, axis=2, keepdims=True)
    dQ = qn - GOP

    ep_list, epl_list, fockp_list = [], [], []
    for b in range(BGEOM):
        ep_b = jnp.dot(G[b], dQ[b], preferred_element_type=jnp.float32)
        epl_b = lax.dot_general(dQ[b], G[b], (((0,), (1,)), ((), ())),
                                preferred_element_type=jnp.float32)
        ep_list.append(ep_b[None])
        epl_list.append(epl_b[None])
    ep = jnp.concatenate(ep_list, axis=0)
    ep_l = jnp.concatenate(epl_list, axis=0)

    couMat = -0.5 * S * (ep + ep_l)
    F = H + couMat
    for b in range(BGEOM):
        M1 = jnp.dot(F[b], phiS[b], preferred_element_type=jnp.float32)
        f1 = _dgT00(phiS[b], M1)
        f2 = _dgT00(M1, phiS[b])
        fockp_list.append((0.5 * (f1 + f2))[None])
    A = jnp.concatenate(fockp_list, axis=0)

    eye, _ = _masks()
    V = jnp.where(eye, jnp.float32(1.0), jnp.float32(0.0))
    V = jnp.broadcast_to(V, A.shape)
    zf0 = jnp.zeros_like(A)
    d0 = jnp.sum(jnp.where(eye, A, zf0), axis=1, keepdims=True)
    A, V, _ = lax.fori_loop(0, 63 * NSWEEP, _jacobi_step, (A, V, d0))

    zf = jnp.zeros_like(A)
    Aeye = jnp.where(eye, A, zf)
    d_l = jnp.sum(Aeye, axis=1, keepdims=True)
    d_s = jnp.sum(Aeye, axis=2, keepdims=True)
    il = lax.broadcasted_iota(jnp.int32, A.shape, 2)
    isub = lax.broadcasted_iota(jnp.int32, A.shape, 1)
    less = jnp.where(d_l < d_s, jnp.float32(1.0), jnp.float32(0.0))
    tie = jnp.where((d_l == d_s) & (il < isub), jnp.float32(1.0),
                    jnp.float32(0.0))
    rank = jnp.sum(less + tie, axis=2, keepdims=True)
    P = jnp.where(rank == il.astype(jnp.float32), jnp.float32(1.0),
                  jnp.float32(0.0))

    orbf_list = []
    for b in range(BGEOM):
        W_b = jnp.dot(V[b], P[b], preferred_element_type=jnp.float32)
        orb_b = jnp.dot(phiS[b], W_b, preferred_element_type=jnp.float32)
        orbf_list.append(orb_b[None])
    orb = jnp.concatenate(orbf_list, axis=0)
    orb_f = occ * orb
    rho_list = []
    for b in range(BGEOM):
        rho_list.append((2.0 * _dg11(orb_f[b], orb_f[b]))[None])
    rho = jnp.concatenate(rho_list, axis=0)

    e1 = jnp.sum(rho * H, axis=2, keepdims=True)
    e1 = jnp.sum(e1, axis=1, keepdims=True)
    e2 = 0.5 * jnp.sum(dQ * ep, axis=1, keepdims=True)
    out_ref[...] = jnp.broadcast_to(e1 + e2, (BGEOM, 1, 128))


def _mat_spec():
    return pl.BlockSpec((BGEOM, BSIZE, BSIZE), lambda g: (g, 0, 0))


def _dense_block(S, G, rho_in, qneutral, phiS, occ_mask, H):
    return pl.pallas_call(
        _dense_body,
        grid=(NGEOM // BGEOM,),
        in_specs=[_mat_spec(), _mat_spec(), _mat_spec(),
                  pl.BlockSpec((BGEOM, BSIZE, 1), lambda g: (g, 0, 0)),
                  _mat_spec(), _mat_spec(), _mat_spec()],
        out_specs=pl.BlockSpec((BGEOM, 1, 128), lambda g: (g, 0, 0)),
        out_shape=jax.ShapeDtypeStruct((NGEOM, 1, 128), jnp.float32),
    )(S, G, rho_in, qneutral, phiS, occ_mask, H)


def kernel(x, net_base, rot_tensor, S, G, rho_in, qneutral, phiS, occ_mask,
           idx_x, gather_rot, gather_oper, gather_rep, seg_rep):
    net_vals = net_base.at[idx_x].set(x)
    vals = net_vals[gather_rot].reshape((-1, 3))[:, :, None]
    rot_out_temp = jnp.matmul(rot_tensor, vals)[:, :, 0]
    rot_out = jnp.concatenate(
        [jnp.array([0.0, 1.0], dtype=jnp.float32), rot_out_temp.ravel()])
    H = rot_out[gather_oper]

    e12 = _dense_block(S, G, rho_in, qneutral, phiS, occ_mask, H)

    Erep = jax.ops.segment_sum(net_vals[gather_rep], seg_rep,
                               num_segments=NGEOM)
    return e12[:, 0, 0] + Erep


# speedup vs baseline: 1.3363x; 1.3363x over previous
import functools

import jax
import jax.numpy as jnp
from jax import lax
from jax.experimental import pallas as pl
from jax.experimental.pallas import tpu as pltpu

NGEOM = 200
BSIZE = 64
NVALS = 2000000
BGEOM = 20
NSWEEP = 4


def _dgT00(a, b):
    return lax.dot_general(a, b, (((0,), (0,)), ((), ())),
                           preferred_element_type=jnp.float32)


def _dg11(a, b):
    return lax.dot_general(a, b, (((1,), (1,)), ((), ())),
                           preferred_element_type=jnp.float32)


def _coeffs(app, aqq, apq):
    zero = apq == 0.0
    apq_s = jnp.where(zero, 1.0, apq)
    tau = (aqq - app) / (2.0 * apq_s)
    t = jnp.sign(tau) / (jnp.abs(tau) + jnp.sqrt(tau * tau + 1.0))
    t = jnp.where(tau == 0.0, 1.0, t)
    t = jnp.where(zero, 0.0, t)
    c = lax.rsqrt(t * t + 1.0)
    s = t * c
    c = jnp.where(zero, 1.0, c)
    s = jnp.where(zero, 0.0, s)
    return c, s, t


def _perm_rows(A):
    return jnp.concatenate([A[:, 0:1], A[:, 32:33], A[:, 1:31],
                            A[:, 33:64], A[:, 31:32]], axis=1)


def _perm_cols(A):
    return jnp.concatenate([A[:, :, 0:1], A[:, :, 32:33], A[:, :, 1:31],
                            A[:, :, 33:64], A[:, :, 31:32]], axis=2)


def _masks():
    ks = lax.broadcasted_iota(jnp.int32, (1, BSIZE, BSIZE), 1)
    js = lax.broadcasted_iota(jnp.int32, (1, BSIZE, BSIZE), 2)
    eye = ks == js
    off = js == ks + 32
    return eye, off


def _rot_perm_rows(A, c_s, s_s):
    T, Bt = A[:, 0:32, :], A[:, 32:64, :]
    up = c_s * T - s_s * Bt
    dn = s_s * T + c_s * Bt
    return jnp.concatenate([up[:, 0:1], dn[:, 0:1], up[:, 1:31],
                            dn[:, 1:32], up[:, 31:32]], axis=1)


def _rot_perm_cols(A, c_l, s_l):
    L, R = A[:, :, 0:32], A[:, :, 32:64]
    lf = c_l * L - s_l * R
    rt = s_l * L + c_l * R
    return jnp.concatenate([lf[:, :, 0:1], rt[:, :, 0:1], lf[:, :, 1:31],
                            rt[:, :, 1:32], lf[:, :, 31:32]], axis=2)


def _perm_lanes(d):
    return jnp.concatenate([d[:, :, 0:1], d[:, :, 32:33], d[:, :, 1:31],
                            d[:, :, 33:64], d[:, :, 31:32]], axis=2)


def _jacobi_step(offf, _, carry):
    A, U, d = carry
    ov_l = jnp.sum(offf * A, axis=1, keepdims=True)
    apq = ov_l[:, :, 32:64]
    app, aqq = d[:, :, 0:32], d[:, :, 32:64]
    c_l, s_l, t_l = _coeffs(app, aqq, apq)
    d_new = jnp.concatenate([app - t_l * apq, aqq + t_l * apq], axis=2)
    d = _perm_lanes(d_new)
    eye32 = (lax.broadcasted_iota(jnp.int32, (32, 32), 0) ==
             lax.broadcasted_iota(jnp.int32, (32, 32), 1)
             ).astype(jnp.float32)
    cs_list = []
    for b in range(BGEOM):
        CS = jnp.concatenate([c_l[b], s_l[b]], axis=0)
        CST = lax.dot_general(eye32, CS, (((1,), (1,)), ((), ())),
                              preferred_element_type=jnp.float32)
        cs_list.append(CST[None])
    CSs = jnp.concatenate(cs_list, axis=0)
    c_s, s_s = CSs[:, :, 0:1], CSs[:, :, 1:2]
    A = _rot_perm_rows(A, c_s, s_s)
    A = _rot_perm_cols(A, c_l, s_l)
    U = _rot_perm_rows(U, c_s, s_s)
    return A, U, d


def _dense_body(S_ref, G_ref, rho_ref, qn_ref, phiS_ref, occ_ref, H_ref,
                out_ref):
    S = S_ref[...]
    G = G_ref[...]
    rho_in = rho_ref[...]
    qn = qn_ref[...]
    phiS = phiS_ref[...]
    occ = occ_ref[...]
    H = H_ref[...]

    qbasis = rho_in * S
    GOP = jnp.sum(qbasis, axis=2, keepdims=True)
    dQ = qn - GOP

    ep_list, epl_list, fockp_list = [], [], []
    for b in range(BGEOM):
        ep_b = jnp.dot(G[b], dQ[b], preferred_element_type=jnp.float32)
        epl_b = lax.dot_general(dQ[b], G[b], (((0,), (1,)), ((), ())),
                                preferred_element_type=jnp.float32)
        ep_list.append(ep_b[None])
        epl_list.append(epl_b[None])
    ep = jnp.concatenate(ep_list, axis=0)
    ep_l = jnp.concatenate(epl_list, axis=0)

    couMat = -0.5 * S * (ep + ep_l)
    F = H + couMat
    for b in range(BGEOM):
        M1 = jnp.dot(F[b], phiS[b], preferred_element_type=jnp.float32)
        f1 = _dgT00(phiS[b], M1)
        f2 = _dgT00(M1, phiS[b])
        fockp_list.append((0.5 * (f1 + f2))[None])
    A = jnp.concatenate(fockp_list, axis=0)

    eye, off = _masks()
    eyef = jnp.where(eye, jnp.float32(1.0), jnp.float32(0.0))
    offf = jnp.where(off, jnp.float32(1.0), jnp.float32(0.0))
    U = jnp.broadcast_to(eyef, A.shape)
    d0 = jnp.sum(eyef * A, axis=1, keepdims=True)
    A, U, _ = lax.fori_loop(0, 63 * NSWEEP,
                            functools.partial(_jacobi_step, offf),
                            (A, U, d0))

    zf = jnp.zeros_like(A)
    Aeye = jnp.where(eye, A, zf)
    d_l = jnp.sum(Aeye, axis=1, keepdims=True)
    d_s = jnp.sum(Aeye, axis=2, keepdims=True)
    il = lax.broadcasted_iota(jnp.int32, A.shape, 2)
    isub = lax.broadcasted_iota(jnp.int32, A.shape, 1)
    less = jnp.where(d_l < d_s, jnp.float32(1.0), jnp.float32(0.0))
    tie = jnp.where((d_l == d_s) & (il < isub), jnp.float32(1.0),
                    jnp.float32(0.0))
    rank = jnp.sum(less + tie, axis=2, keepdims=True)
    P = jnp.where(rank == il.astype(jnp.float32), jnp.float32(1.0),
                  jnp.float32(0.0))

    orbf_list = []
    for b in range(BGEOM):
        phiV_b = _dg11(phiS[b], U[b])
        orb_b = jnp.dot(phiV_b, P[b], preferred_element_type=jnp.float32)
        orbf_list.append(orb_b[None])
    orb = jnp.concatenate(orbf_list, axis=0)
    orb_f = occ * orb
    rho_list = []
    for b in range(BGEOM):
        rho_list.append((2.0 * _dg11(orb_f[b], orb_f[b]))[None])
    rho = jnp.concatenate(rho_list, axis=0)

    e1 = jnp.sum(rho * H, axis=2, keepdims=True)
    e1 = jnp.sum(e1, axis=1, keepdims=True)
    e2 = 0.5 * jnp.sum(dQ * ep, axis=1, keepdims=True)
    out_ref[...] = jnp.broadcast_to(e1 + e2, (BGEOM, 1, 128))


def _mat_spec():
    return pl.BlockSpec((BGEOM, BSIZE, BSIZE), lambda g: (g, 0, 0))


def _dense_block(S, G, rho_in, qneutral, phiS, occ_mask, H):
    return pl.pallas_call(
        _dense_body,
        grid=(NGEOM // BGEOM,),
        in_specs=[_mat_spec(), _mat_spec(), _mat_spec(),
                  pl.BlockSpec((BGEOM, BSIZE, 1), lambda g: (g, 0, 0)),
                  _mat_spec(), _mat_spec(), _mat_spec()],
        out_specs=pl.BlockSpec((BGEOM, 1, 128), lambda g: (g, 0, 0)),
        out_shape=jax.ShapeDtypeStruct((NGEOM, 1, 128), jnp.float32),
    )(S, G, rho_in, qneutral, phiS, occ_mask, H)


def kernel(x, net_base, rot_tensor, S, G, rho_in, qneutral, phiS, occ_mask,
           idx_x, gather_rot, gather_oper, gather_rep, seg_rep):
    net_vals = net_base.at[idx_x].set(x)
    vals = net_vals[gather_rot].reshape((-1, 3))[:, :, None]
    rot_out_temp = jnp.matmul(rot_tensor, vals)[:, :, 0]
    rot_out = jnp.concatenate(
        [jnp.array([0.0, 1.0], dtype=jnp.float32), rot_out_temp.ravel()])
    H = rot_out[gather_oper]

    e12 = _dense_block(S, G, rho_in, qneutral, phiS, occ_mask, H)

    Erep = jax.ops.segment_sum(net_vals[gather_rep], seg_rep,
                               num_segments=NGEOM)
    return e12[:, 0, 0] + Erep


# speedup vs baseline: 1.6211x; 1.2131x over previous
import functools

import jax
import jax.numpy as jnp
from jax import lax
from jax.experimental import pallas as pl
from jax.experimental.pallas import tpu as pltpu

NGEOM = 200
BSIZE = 64
NVALS = 2000000
BGEOM = 20
NSWEEP = 3


def _dgT00(a, b):
    return lax.dot_general(a, b, (((0,), (0,)), ((), ())),
                           preferred_element_type=jnp.float32)


def _dg11(a, b):
    return lax.dot_general(a, b, (((1,), (1,)), ((), ())),
                           preferred_element_type=jnp.float32)


def _coeffs(app, aqq, apq):
    zero = apq == 0.0
    apq_s = jnp.where(zero, 1.0, apq)
    tau = (aqq - app) / (2.0 * apq_s)
    t = jnp.sign(tau) / (jnp.abs(tau) + jnp.sqrt(tau * tau + 1.0))
    t = jnp.where(tau == 0.0, 1.0, t)
    t = jnp.where(zero, 0.0, t)
    c = lax.rsqrt(t * t + 1.0)
    s = t * c
    c = jnp.where(zero, 1.0, c)
    s = jnp.where(zero, 0.0, s)
    return c, s, t


def _perm_rows(A):
    return jnp.concatenate([A[:, 0:1], A[:, 32:33], A[:, 1:31],
                            A[:, 33:64], A[:, 31:32]], axis=1)


def _perm_cols(A):
    return jnp.concatenate([A[:, :, 0:1], A[:, :, 32:33], A[:, :, 1:31],
                            A[:, :, 33:64], A[:, :, 31:32]], axis=2)


def _masks():
    ks = lax.broadcasted_iota(jnp.int32, (1, BSIZE, BSIZE), 1)
    js = lax.broadcasted_iota(jnp.int32, (1, BSIZE, BSIZE), 2)
    eye = ks == js
    off = js == ks + 32
    return eye, off


def _rot_perm_rows(A, c_s, s_s):
    T, Bt = A[:, 0:32, :], A[:, 32:64, :]
    up = c_s * T - s_s * Bt
    dn = s_s * T + c_s * Bt
    return jnp.concatenate([up[:, 0:1], dn[:, 0:1], up[:, 1:31],
                            dn[:, 1:32], up[:, 31:32]], axis=1)


def _rot_perm_cols(A, c_l, s_l):
    L, R = A[:, :, 0:32], A[:, :, 32:64]
    lf = c_l * L - s_l * R
    rt = s_l * L + c_l * R
    return jnp.concatenate([lf[:, :, 0:1], rt[:, :, 0:1], lf[:, :, 1:31],
                            rt[:, :, 1:32], lf[:, :, 31:32]], axis=2)


def _perm_lanes(d):
    return jnp.concatenate([d[:, :, 0:1], d[:, :, 32:33], d[:, :, 1:31],
                            d[:, :, 33:64], d[:, :, 31:32]], axis=2)


def _jacobi_step(offf, _, carry):
    A, U, d = carry
    ov_l = jnp.sum(offf * A, axis=1, keepdims=True)
    apq = ov_l[:, :, 32:64]
    app, aqq = d[:, :, 0:32], d[:, :, 32:64]
    c_l, s_l, t_l = _coeffs(app, aqq, apq)
    d_new = jnp.concatenate([app - t_l * apq, aqq + t_l * apq], axis=2)
    d = _perm_lanes(d_new)
    eye32 = (lax.broadcasted_iota(jnp.int32, (32, 32), 0) ==
             lax.broadcasted_iota(jnp.int32, (32, 32), 1)
             ).astype(jnp.float32)
    cs_list = []
    for b in range(BGEOM):
        CS = jnp.concatenate([c_l[b], s_l[b]], axis=0)
        CST = lax.dot_general(eye32, CS, (((1,), (1,)), ((), ())),
                              preferred_element_type=jnp.float32)
        cs_list.append(CST[None])
    CSs = jnp.concatenate(cs_list, axis=0)
    c_s, s_s = CSs[:, :, 0:1], CSs[:, :, 1:2]
    A = _rot_perm_rows(A, c_s, s_s)
    A = _rot_perm_cols(A, c_l, s_l)
    U = _rot_perm_rows(U, c_s, s_s)
    return A, U, d


def _dense_body(S_ref, G_ref, rho_ref, qn_ref, phiS_ref, occ_ref, H_ref,
                out_ref):
    S = S_ref[...]
    G = G_ref[...]
    rho_in = rho_ref[...]
    qn = qn_ref[...]
    phiS = phiS_ref[...]
    occ = occ_ref[...]
    H = H_ref[...]

    qbasis = rho_in * S
    GOP = jnp.sum(qbasis, axis=2, keepdims=True)
    dQ = qn - GOP

    ep_list, epl_list, fockp_list = [], [], []
    for b in range(BGEOM):
        ep_b = jnp.dot(G[b], dQ[b], preferred_element_type=jnp.float32)
        epl_b = lax.dot_general(dQ[b], G[b], (((0,), (1,)), ((), ())),
                                preferred_element_type=jnp.float32)
        ep_list.append(ep_b[None])
        epl_list.append(epl_b[None])
    ep = jnp.concatenate(ep_list, axis=0)
    ep_l = jnp.concatenate(epl_list, axis=0)

    couMat = -0.5 * S * (ep + ep_l)
    F = H + couMat
    for b in range(BGEOM):
        M1 = jnp.dot(F[b], phiS[b], preferred_element_type=jnp.float32)
        f1 = _dgT00(phiS[b], M1)
        f2 = _dgT00(M1, phiS[b])
        fockp_list.append((0.5 * (f1 + f2))[None])
    A = jnp.concatenate(fockp_list, axis=0)

    eye, off = _masks()
    eyef = jnp.where(eye, jnp.float32(1.0), jnp.float32(0.0))
    offf = jnp.where(off, jnp.float32(1.0), jnp.float32(0.0))
    U = jnp.broadcast_to(eyef, A.shape)
    d0 = jnp.sum(eyef * A, axis=1, keepdims=True)
    A, U, _ = lax.fori_loop(0, 63 * NSWEEP,
                            functools.partial(_jacobi_step, offf),
                            (A, U, d0))

    zf = jnp.zeros_like(A)
    Aeye = jnp.where(eye, A, zf)
    d_l = jnp.sum(Aeye, axis=1, keepdims=True)
    d_s = jnp.sum(Aeye, axis=2, keepdims=True)
    il = lax.broadcasted_iota(jnp.int32, A.shape, 2)
    isub = lax.broadcasted_iota(jnp.int32, A.shape, 1)
    less = jnp.where(d_l < d_s, jnp.float32(1.0), jnp.float32(0.0))
    tie = jnp.where((d_l == d_s) & (il < isub), jnp.float32(1.0),
                    jnp.float32(0.0))
    rank = jnp.sum(less + tie, axis=2, keepdims=True)
    P = jnp.where(rank == il.astype(jnp.float32), jnp.float32(1.0),
                  jnp.float32(0.0))

    orbf_list = []
    for b in range(BGEOM):
        phiV_b = _dg11(phiS[b], U[b])
        orb_b = jnp.dot(phiV_b, P[b], preferred_element_type=jnp.float32)
        orbf_list.append(orb_b[None])
    orb = jnp.concatenate(orbf_list, axis=0)
    orb_f = occ * orb
    rho_list = []
    for b in range(BGEOM):
        rho_list.append((2.0 * _dg11(orb_f[b], orb_f[b]))[None])
    rho = jnp.concatenate(rho_list, axis=0)

    e1 = jnp.sum(rho * H, axis=2, keepdims=True)
    e1 = jnp.sum(e1, axis=1, keepdims=True)
    e2 = 0.5 * jnp.sum(dQ * ep, axis=1, keepdims=True)
    out_ref[...] = jnp.broadcast_to(e1 + e2, (BGEOM, 1, 128))


def _mat_spec():
    return pl.BlockSpec((BGEOM, BSIZE, BSIZE), lambda g: (g, 0, 0))


def _dense_block(S, G, rho_in, qneutral, phiS, occ_mask, H):
    return pl.pallas_call(
        _dense_body,
        grid=(NGEOM // BGEOM,),
        in_specs=[_mat_spec(), _mat_spec(), _mat_spec(),
                  pl.BlockSpec((BGEOM, BSIZE, 1), lambda g: (g, 0, 0)),
                  _mat_spec(), _mat_spec(), _mat_spec()],
        out_specs=pl.BlockSpec((BGEOM, 1, 128), lambda g: (g, 0, 0)),
        out_shape=jax.ShapeDtypeStruct((NGEOM, 1, 128), jnp.float32),
    )(S, G, rho_in, qneutral, phiS, occ_mask, H)


def kernel(x, net_base, rot_tensor, S, G, rho_in, qneutral, phiS, occ_mask,
           idx_x, gather_rot, gather_oper, gather_rep, seg_rep):
    net_vals = net_base.at[idx_x].set(x)
    vals = net_vals[gather_rot].reshape((-1, 3))[:, :, None]
    rot_out_temp = jnp.matmul(rot_tensor, vals)[:, :, 0]
    rot_out = jnp.concatenate(
        [jnp.array([0.0, 1.0], dtype=jnp.float32), rot_out_temp.ravel()])
    H = rot_out[gather_oper]

    e12 = _dense_block(S, G, rho_in, qneutral, phiS, occ_mask, H)

    Erep = jax.ops.segment_sum(net_vals[gather_rep], seg_rep,
                               num_segments=NGEOM)
    return e12[:, 0, 0] + Erep


# speedup vs baseline: 1.6293x; 1.0051x over previous
import functools

import jax
import jax.numpy as jnp
from jax import lax
from jax.experimental import pallas as pl
from jax.experimental.pallas import tpu as pltpu
from jax.experimental.pallas import tpu_sc as plsc

NGEOM = 200
BSIZE = 64
NVALS = 2000000
BGEOM = 20
NSWEEP = 3


def _dgT00(a, b):
    return lax.dot_general(a, b, (((0,), (0,)), ((), ())),
                           preferred_element_type=jnp.float32)


def _dg11(a, b):
    return lax.dot_general(a, b, (((1,), (1,)), ((), ())),
                           preferred_element_type=jnp.float32)


def _coeffs(app, aqq, apq):
    zero = apq == 0.0
    apq_s = jnp.where(zero, 1.0, apq)
    tau = (aqq - app) / (2.0 * apq_s)
    t = jnp.sign(tau) / (jnp.abs(tau) + jnp.sqrt(tau * tau + 1.0))
    t = jnp.where(tau == 0.0, 1.0, t)
    t = jnp.where(zero, 0.0, t)
    c = lax.rsqrt(t * t + 1.0)
    s = t * c
    c = jnp.where(zero, 1.0, c)
    s = jnp.where(zero, 0.0, s)
    return c, s, t


def _perm_rows(A):
    return jnp.concatenate([A[:, 0:1], A[:, 32:33], A[:, 1:31],
                            A[:, 33:64], A[:, 31:32]], axis=1)


def _perm_cols(A):
    return jnp.concatenate([A[:, :, 0:1], A[:, :, 32:33], A[:, :, 1:31],
                            A[:, :, 33:64], A[:, :, 31:32]], axis=2)


def _masks():
    ks = lax.broadcasted_iota(jnp.int32, (1, BSIZE, BSIZE), 1)
    js = lax.broadcasted_iota(jnp.int32, (1, BSIZE, BSIZE), 2)
    eye = ks == js
    off = js == ks + 32
    return eye, off


def _rot_perm_rows(A, c_s, s_s):
    T, Bt = A[:, 0:32, :], A[:, 32:64, :]
    up = c_s * T - s_s * Bt
    dn = s_s * T + c_s * Bt
    return jnp.concatenate([up[:, 0:1], dn[:, 0:1], up[:, 1:31],
                            dn[:, 1:32], up[:, 31:32]], axis=1)


def _rot_perm_cols(A, c_l, s_l):
    L, R = A[:, :, 0:32], A[:, :, 32:64]
    lf = c_l * L - s_l * R
    rt = s_l * L + c_l * R
    return jnp.concatenate([lf[:, :, 0:1], rt[:, :, 0:1], lf[:, :, 1:31],
                            rt[:, :, 1:32], lf[:, :, 31:32]], axis=2)


def _perm_lanes(d):
    return jnp.concatenate([d[:, :, 0:1], d[:, :, 32:33], d[:, :, 1:31],
                            d[:, :, 33:64], d[:, :, 31:32]], axis=2)


def _jacobi_step(offf, _, carry):
    A, U, d = carry
    ov_l = jnp.sum(offf * A, axis=1, keepdims=True)
    apq = ov_l[:, :, 32:64]
    app, aqq = d[:, :, 0:32], d[:, :, 32:64]
    c_l, s_l, t_l = _coeffs(app, aqq, apq)
    d_new = jnp.concatenate([app - t_l * apq, aqq + t_l * apq], axis=2)
    d = _perm_lanes(d_new)
    eye32 = (lax.broadcasted_iota(jnp.int32, (32, 32), 0) ==
             lax.broadcasted_iota(jnp.int32, (32, 32), 1)
             ).astype(jnp.float32)
    cs_list = []
    for b in range(BGEOM):
        CS = jnp.concatenate([c_l[b], s_l[b]], axis=0)
        CST = lax.dot_general(eye32, CS, (((1,), (1,)), ((), ())),
                              preferred_element_type=jnp.float32)
        cs_list.append(CST[None])
    CSs = jnp.concatenate(cs_list, axis=0)
    c_s, s_s = CSs[:, :, 0:1], CSs[:, :, 1:2]
    A = _rot_perm_rows(A, c_s, s_s)
    A = _rot_perm_cols(A, c_l, s_l)
    U = _rot_perm_rows(U, c_s, s_s)
    return A, U, d


def _dense_body(S_ref, G_ref, rho_ref, qn_ref, phiS_ref, occ_ref, H_ref,
                out_ref):
    S = S_ref[...]
    G = G_ref[...]
    rho_in = rho_ref[...]
    qn = qn_ref[...]
    phiS = phiS_ref[...]
    occ = occ_ref[...]
    H = H_ref[...]

    qbasis = rho_in * S
    GOP = jnp.sum(qbasis, axis=2, keepdims=True)
    dQ = qn - GOP

    ep_list, epl_list, fockp_list = [], [], []
    for b in range(BGEOM):
        ep_b = jnp.dot(G[b], dQ[b], preferred_element_type=jnp.float32)
        epl_b = lax.dot_general(dQ[b], G[b], (((0,), (1,)), ((), ())),
                                preferred_element_type=jnp.float32)
        ep_list.append(ep_b[None])
        epl_list.append(epl_b[None])
    ep = jnp.concatenate(ep_list, axis=0)
    ep_l = jnp.concatenate(epl_list, axis=0)

    couMat = -0.5 * S * (ep + ep_l)
    F = H + couMat
    for b in range(BGEOM):
        M1 = jnp.dot(F[b], phiS[b], preferred_element_type=jnp.float32)
        f1 = _dgT00(phiS[b], M1)
        f2 = _dgT00(M1, phiS[b])
        fockp_list.append((0.5 * (f1 + f2))[None])
    A = jnp.concatenate(fockp_list, axis=0)

    eye, off = _masks()
    eyef = jnp.where(eye, jnp.float32(1.0), jnp.float32(0.0))
    offf = jnp.where(off, jnp.float32(1.0), jnp.float32(0.0))
    U = jnp.broadcast_to(eyef, A.shape)
    d0 = jnp.sum(eyef * A, axis=1, keepdims=True)
    A, U, _ = lax.fori_loop(0, 63 * NSWEEP,
                            functools.partial(_jacobi_step, offf),
                            (A, U, d0))

    zf = jnp.zeros_like(A)
    Aeye = jnp.where(eye, A, zf)
    d_l = jnp.sum(Aeye, axis=1, keepdims=True)
    d_s = jnp.sum(Aeye, axis=2, keepdims=True)
    il = lax.broadcasted_iota(jnp.int32, A.shape, 2)
    isub = lax.broadcasted_iota(jnp.int32, A.shape, 1)
    less = jnp.where(d_l < d_s, jnp.float32(1.0), jnp.float32(0.0))
    tie = jnp.where((d_l == d_s) & (il < isub), jnp.float32(1.0),
                    jnp.float32(0.0))
    rank = jnp.sum(less + tie, axis=2, keepdims=True)
    P = jnp.where(rank == il.astype(jnp.float32), jnp.float32(1.0),
                  jnp.float32(0.0))

    orbf_list = []
    for b in range(BGEOM):
        phiV_b = _dg11(phiS[b], U[b])
        orb_b = jnp.dot(phiV_b, P[b], preferred_element_type=jnp.float32)
        orbf_list.append(orb_b[None])
    orb = jnp.concatenate(orbf_list, axis=0)
    orb_f = occ * orb
    rho_list = []
    for b in range(BGEOM):
        rho_list.append((2.0 * _dg11(orb_f[b], orb_f[b]))[None])
    rho = jnp.concatenate(rho_list, axis=0)

    e1 = jnp.sum(rho * H, axis=2, keepdims=True)
    e1 = jnp.sum(e1, axis=1, keepdims=True)
    e2 = 0.5 * jnp.sum(dQ * ep, axis=1, keepdims=True)
    out_ref[...] = jnp.broadcast_to(e1 + e2, (BGEOM, 1, 128))


def _mat_spec():
    return pl.BlockSpec((BGEOM, BSIZE, BSIZE), lambda g: (g, 0, 0))


def _dense_block(S, G, rho_in, qneutral, phiS, occ_mask, H):
    return pl.pallas_call(
        _dense_body,
        grid=(NGEOM // BGEOM,),
        in_specs=[_mat_spec(), _mat_spec(), _mat_spec(),
                  pl.BlockSpec((BGEOM, BSIZE, 1), lambda g: (g, 0, 0)),
                  _mat_spec(), _mat_spec(), _mat_spec()],
        out_specs=pl.BlockSpec((BGEOM, 1, 128), lambda g: (g, 0, 0)),
        out_shape=jax.ShapeDtypeStruct((NGEOM, 1, 128), jnp.float32),
    )(S, G, rho_in, qneutral, phiS, occ_mask, H)


def _sc_mesh():
    return plsc.VectorSubcoreMesh(core_axis_name="c", subcore_axis_name="s")


def _sc_wid():
    return lax.axis_index("s") * 2 + lax.axis_index("c")


def _sc_net_vals(net_base, x, idx_x):
    npad = 32 * 3200 - 100000
    xp = jnp.concatenate([x, jnp.broadcast_to(x[0:1], (npad,))]
                         ).reshape(32, 25, 128)
    ip = jnp.concatenate([idx_x, jnp.broadcast_to(idx_x[0:1], (npad,))]
                         ).reshape(32, 25, 128)

    @functools.partial(
        pl.kernel, mesh=_sc_mesh(),
        out_type=jax.ShapeDtypeStruct((NVALS,), jnp.float32),
        scratch_types=[pltpu.VMEM((80016,), jnp.float32),
                       pltpu.VMEM((25, 128), jnp.float32),
                       pltpu.VMEM((25, 128), jnp.int32)],
    )
    def k(base_hbm, x_hbm, idx_hbm, out_hbm, buf, xbuf, ibuf):
        wid = _sc_wid()

        @pl.when(wid < 25)
        def _():
            lo = wid * 80000
            pltpu.sync_copy(
                base_hbm.at[pl.ds(pl.multiple_of(lo, 8), 80000)],
                buf.at[pl.ds(0, 80000)])

            def piece(p, c0):
                pltpu.sync_copy(x_hbm.at[p], xbuf)
                pltpu.sync_copy(idx_hbm.at[p], ibuf)

                def inner(it, c1):
                    rr = it // 8
                    ii = it - rr * 8
                    iv = ibuf[rr, pl.ds(ii * 16, 16)]
                    xv = xbuf[rr, pl.ds(ii * 16, 16)]
                    loc = iv - lo
                    ok = (loc >= 0) & (loc < 80000)
                    loc2 = jnp.where(ok, loc, 80000)
                    plsc.store_scatter(buf, [loc2], xv)
                    return c1
                return lax.fori_loop(0, 200, inner, c0)
            lax.fori_loop(0, 32, piece, 0)
            pltpu.sync_copy(buf.at[pl.ds(0, 80000)],
                            out_hbm.at[pl.ds(pl.multiple_of(lo, 8),
                                             80000)])

    return k(net_base, xp, ip)


def _sc_rot(net_flat, gather_rot, rot_tensor):
    rt_flat = rot_tensor.reshape(1800000)

    @functools.partial(
        pl.kernel, mesh=_sc_mesh(),
        out_type=jax.ShapeDtypeStruct((604000,), jnp.float32),
        scratch_types=[pltpu.VMEM((24000,), jnp.int32),
                       pltpu.VMEM((36000,), jnp.float32),
                       pltpu.VMEM((3, 32, 128), jnp.int32),
                       pltpu.VMEM((3, 32, 128), jnp.float32),
                       pltpu.VMEM((12000,), jnp.float32),
                       pltpu.VMEM((16,), jnp.float32),
                       pltpu.SemaphoreType.DMA],
    )
    def k(nv_hbm, gr_hbm, rt_hbm, out_hbm, grbuf, rtbuf, idxb, valb, outb,
          sbuf, sem):
        wid = _sc_wid()
        iota = lax.iota(jnp.int32, 16)

        @pl.when(wid == 0)
        def _():
            sbuf[pl.ds(0, 16)] = jnp.where(iota == 1, 1.0, 0.0
                                           ).astype(jnp.float32)
            pltpu.sync_copy(sbuf, out_hbm.at[pl.ds(0, 16)])

        @pl.when(wid < 25)
        def _():
            pltpu.sync_copy(
                gr_hbm.at[pl.ds(pl.multiple_of(wid * 24000, 8), 24000)],
                grbuf)
            for hh in range(2):
                pltpu.sync_copy(
                    rt_hbm.at[pl.ds(
                        pl.multiple_of(wid * 72000 + hh * 36000, 8),
                        36000)],
                    rtbuf)

                def build(it, c):
                    r = it // 8
                    i = it - r * 8
                    mc = jnp.minimum(r * 128 + i * 16 + iota, 3999)
                    for kk in range(3):
                        g = plsc.load_gather(
                            grbuf, [hh * 12000 + 3 * mc + kk])
                        idxb[kk, r, pl.ds(i * 16, 16)] = g
                    return c
                lax.fori_loop(0, 256, build, 0)

                def fire(bb, c):
                    dmas = []
                    for kk in range(3):
                        for rr in range(4):
                            r = bb * 4 + rr
                            dmas.append(pltpu.async_copy(
                                nv_hbm.at[idxb.at[kk, r]],
                                valb.at[kk, r], sem))
                    for d in dmas:
                        d.wait()
                    return c
                lax.fori_loop(0, 8, fire, 0)

                def comp(it, c):
                    r = it // 8
                    i = it - r * 8
                    mc = jnp.minimum(r * 128 + i * 16 + iota, 3999)
                    pos = it * 16
                    vs = [valb[kk, r, pl.ds(i * 16, 16)]
                          for kk in range(3)]
                    for j in range(3):
                        acc = jnp.zeros((16,), jnp.float32)
                        for kk in range(3):
                            rt = plsc.load_gather(
                                rtbuf, [9 * mc + (3 * j + kk)])
                            acc = acc + rt * vs[kk]
                        outb[pl.ds(j * 4000 + pos, 16)] = acc
                    return c
                lax.fori_loop(0, 250, comp, 0)

                for j in range(3):
                    row = 1 + j * 50 + wid * 2 + hh
                    pltpu.sync_copy(
                        outb.at[pl.ds(j * 4000, 4000)],
                        out_hbm.at[pl.ds(pl.multiple_of(row * 4000, 8),
                                         4000)])

    return k(net_flat, gather_rot, rt_flat)


def _sc_hgather(rot_out, gather_oper):
    g2 = gather_oper.reshape(32, 200, 128)

    @functools.partial(
        pl.kernel, mesh=_sc_mesh(),
        out_type=jax.ShapeDtypeStruct((32, 200, 128), jnp.float32),
        scratch_types=[pltpu.VMEM((200, 128), jnp.int32),
                       pltpu.VMEM((200, 128), jnp.float32),
                       pltpu.SemaphoreType.DMA],
    )
    def k(rc_hbm, g_hbm, out_hbm, g2buf, hbuf, sem):
        wid = _sc_wid()
        pltpu.sync_copy(g_hbm.at[wid], g2buf)

        def fire(bb, c):
            dmas = []
            for rr in range(8):
                r = bb * 8 + rr
                dmas.append(pltpu.async_copy(
                    rc_hbm.at[g2buf.at[r]], hbuf.at[r], sem))
            for d in dmas:
                d.wait()
            return c
        lax.fori_loop(0, 25, fire, 0)
        pltpu.sync_copy(hbuf, out_hbm.at[wid])

    return k(rot_out, g2).reshape(NGEOM, BSIZE, BSIZE)


def _sc_erep(net_flat, gather_rep, seg_rep):
    grep_pad = jnp.pad(gather_rep, (0, 12000)).reshape(25, 160, 128)

    @functools.partial(
        pl.kernel, mesh=_sc_mesh(),
        out_type=jax.ShapeDtypeStruct((25, 2, 128), jnp.float32),
        scratch_types=[pltpu.VMEM((160, 128), jnp.int32),
                       pltpu.VMEM((160, 128), jnp.float32),
                       pltpu.VMEM((20000,), jnp.int32),
                       pltpu.VMEM((2, 128), jnp.float32),
                       pltpu.SemaphoreType.DMA],
    )
    def k(nv_hbm, gi_hbm, seg_hbm, out_hbm, idxb, valb, segbuf, acc, sem):
        wid = _sc_wid()

        @pl.when(wid < 25)
        def _():
            pltpu.sync_copy(gi_hbm.at[wid], idxb)
            pltpu.sync_copy(
                seg_hbm.at[pl.ds(pl.multiple_of(wid * 20000, 8), 20000)],
                segbuf)
            for rr in range(2):
                for t in range(8):
                    acc[rr, pl.ds(t * 16, 16)] = jnp.zeros((16,),
                                                           jnp.float32)

            def fire(bb, c):
                dmas = []
                for rr in range(8):
                    r = bb * 8 + rr
                    dmas.append(pltpu.async_copy(
                        nv_hbm.at[idxb.at[r]], valb.at[r], sem))
                for d in dmas:
                    d.wait()
                return c
            lax.fori_loop(0, 20, fire, 0)

            iota = lax.iota(jnp.int32, 16)

            def vreg(it, c):
                r = it // 8
                i = it - r * 8
                sv = segbuf[pl.ds(it * 16, 16)]
                vv = valb[r, pl.ds(i * 16, 16)]
                s0 = sv[0]
                s15 = sv[15]

                def seg_add(s, c2):
                    tot = jnp.sum(vv * (sv == s).astype(jnp.float32))
                    base = (s // 16) * 16
                    ar = base // 128
                    ao = base - ar * 128
                    av = acc[ar, pl.ds(ao, 16)]
                    acc[ar, pl.ds(ao, 16)] = av + jnp.where(
                        iota == s - base, tot, jnp.float32(0.0))
                    return c2
                lax.fori_loop(s0, s15 + 1, seg_add, 0)
                return c
            lax.fori_loop(0, 1250, vreg, 0)
            pltpu.sync_copy(acc, out_hbm.at[wid])

    return k(net_flat, grep_pad, seg_rep)


def kernel(x, net_base, rot_tensor, S, G, rho_in, qneutral, phiS, occ_mask,
           idx_x, gather_rot, gather_oper, gather_rep, seg_rep):
    net_vals = net_base.at[idx_x].set(x)
    vals = net_vals[gather_rot].reshape((-1, 3))[:, :, None]
    rot_out_temp = jnp.matmul(rot_tensor, vals)[:, :, 0]
    rot_out = jnp.concatenate(
        [jnp.array([0.0, 1.0], dtype=jnp.float32), rot_out_temp.ravel()])
    H = _sc_hgather(rot_out, gather_oper)
    e12 = _dense_block(S, G, rho_in, qneutral, phiS, occ_mask, H)
    Erep = jax.ops.segment_sum(net_vals[gather_rep], seg_rep,
                               num_segments=NGEOM)
    return e12[:, 0, 0] + Erep
